# baseline (speedup 1.0000x reference)
import numpy as np
import jax
import jax.numpy as jnp
from jax.experimental import pallas as pl
from jax.experimental.pallas import tpu as pltpu

_N = 2048
_H = 1024
_KP = 520
_C = 128
_BB = 2
_NC = _BB * _C

_K_LO = 716

_IDX = np.float32(0.7) * np.float32(1024.0)
_LW = np.float32(np.float32(717.0) - _IDX)
_HW = np.float32(_IDX - np.float32(716.0))

_HI = jax.lax.Precision.HIGHEST


def _build_mats():
    k = np.arange(_KP, dtype=np.int64)[:, None]
    n = np.arange(_H, dtype=np.int64)[None, :]
    ang_e = (2.0 * np.pi / _N) * ((k * (2 * n)) % _N).astype(np.float64)
    ang_o = (2.0 * np.pi / _N) * ((k * (2 * n + 1)) % _N).astype(np.float64)
    valid = (k <= 512).astype(np.float64)
    me = np.concatenate([np.cos(ang_e) * valid, -np.sin(ang_e) * valid], 0)
    mo = np.concatenate([np.cos(ang_o) * valid, -np.sin(ang_o) * valid], 0)
    wgt = np.full((_KP, 1), 2.0 / _N)
    wgt[0] = 1.0 / _N
    wgt[512] = 1.0 / _N
    wgt[513:] = 0.0
    jc = (wgt * np.cos(ang_e)).T
    js = (-(wgt * np.sin(ang_e))).T
    jco = (wgt * np.cos(ang_o)).T
    jso = (-(wgt * np.sin(ang_o))).T
    f32 = np.float32
    return (me.astype(f32), mo.astype(f32), jc.astype(f32), js.astype(f32),
            jco.astype(f32), jso.astype(f32))


_ME, _MO, _JC, _JS, _JCO, _JSO = _build_mats()

import ml_dtypes as _mld


def _split_np(m):
    hi = m.astype(_mld.bfloat16)
    lo = (m - hi.astype(np.float32)).astype(_mld.bfloat16)
    return hi, lo


_MEH, _MEL = _split_np(_ME)
_MOH, _MOL = _split_np(_MO)
_JCH, _JCL = _split_np(_JC)
_JSH, _JSL = _split_np(_JS)
_JCOH, _JCOL = _split_np(_JCO)
_JSOH, _JSOL = _split_np(_JSO)


def _split(x):
    hi = x.astype(jnp.bfloat16)
    lo = (x - hi.astype(jnp.float32)).astype(jnp.bfloat16)
    return hi, lo


def _dot3(mh, ml, x):
    xh, xl = _split(x)
    f = jnp.float32
    return (jnp.dot(mh, xh, preferred_element_type=f) +
            (jnp.dot(mh, xl, preferred_element_type=f) +
             jnp.dot(ml, xh, preferred_element_type=f)))


def _dot3r(x, wh, wl):
    xh, xl = _split(x)
    f = jnp.float32
    return (jnp.dot(xh, wh, preferred_element_type=f) +
            (jnp.dot(xl, wh, preferred_element_type=f) +
             jnp.dot(xh, wl, preferred_element_type=f)))


def _body(x_ref, meh_ref, mel_ref, moh_ref, mol_ref,
          jch_ref, jcl_ref, jsh_ref, jsl_ref,
          jcoh_ref, jcol_ref, jsoh_ref, jsol_ref,
          mr_ref, mi_ref, w1_ref, b1_ref, w2_ref, b2_ref, o_ref):
    xe = jnp.concatenate([x_ref[i, :, :_C] for i in range(_BB)], axis=1)
    xo = jnp.concatenate([x_ref[i, :, _C:] for i in range(_BB)], axis=1)
    e = _dot3(meh_ref[...], mel_ref[...], xe)
    op = _dot3(moh_ref[...], mol_ref[...], xo)
    xlo = e + op
    d = e - op
    xlr, xli = xlo[:_KP], xlo[_KP:]
    dr, di = d[:_KP], d[_KP:]

    riota = jax.lax.broadcasted_iota(jnp.int32, (_KP, _NC), 0)
    inf = jnp.float32(jnp.inf)
    mag_lo = jnp.sqrt(xlr * xlr + xli * xli)
    mag_d = jnp.sqrt(dr * dr + di * di)
    mag_lo = jnp.where(riota > 512, inf, mag_lo)
    mag_d = jnp.where(riota > 511, inf, mag_d)
    smag = jnp.concatenate([mag_lo, mag_d], axis=0)

    bits = jax.lax.bitcast_convert_type(smag, jnp.int32)
    ans = jnp.zeros((1, _NC), jnp.int32)
    for b in range(30, -1, -1):
        t = ans | (1 << b)
        cnt = jnp.sum((bits < t).astype(jnp.int32), axis=0, keepdims=True)
        ans = jnp.where(cnt <= _K_LO, t, ans)
    s_lo = jax.lax.bitcast_convert_type(ans, jnp.float32)
    cnt_le = jnp.sum((bits <= ans).astype(jnp.int32), axis=0, keepdims=True)
    above = jnp.where(smag > s_lo, smag, inf)
    s_hi = jnp.where(cnt_le >= _K_LO + 2, s_lo,
                     jnp.min(above, axis=0, keepdims=True))
    q = s_lo * _LW + s_hi * _HW

    tr = mr_ref[...]
    ti = mi_ref[...]
    cond_lo = mag_lo < q
    cond_d = mag_d < q
    clr = jnp.where(cond_lo, tr, xlr)
    cli = jnp.where(cond_lo, ti, xli)
    cdr = jnp.where(cond_d, tr, dr)
    cdi = jnp.where(cond_d, -ti, di)
    ar = clr + cdr
    ai = cli + cdi
    gr = clr - cdr
    gi = cli - cdi
    is512 = riota == 512
    zero = jnp.float32(0.0)
    ar = jnp.where(is512, 2.0 * clr, ar)
    ai = jnp.where(is512, zero, ai)
    gr = jnp.where(is512, zero, gr)
    gi = jnp.where(is512, 2.0 * cli, gi)

    dxe = (_dot3(jch_ref[...], jcl_ref[...], ar) +
           _dot3(jsh_ref[...], jsl_ref[...], ai))
    dxo = (_dot3(jcoh_ref[...], jcol_ref[...], gr) +
           _dot3(jsoh_ref[...], jsol_ref[...], gi))

    w1h, w1l = _split(w1_ref[...])
    w2h, w2l = _split(w2_ref[...])

    dall = jnp.concatenate(
        [dxe[:, i * _C:(i + 1) * _C] for i in range(_BB)] +
        [dxo[:, i * _C:(i + 1) * _C] for i in range(_BB)], axis=0)
    h = _dot3r(dall, w1h, w1l) + b1_ref[...]
    h = 0.5 * h * (1.0 + jax.lax.erf(h * np.float32(1.0 / np.sqrt(2.0))))
    r = _dot3r(h, w2h, w2l) + b2_ref[...]
    rec = jax.nn.sigmoid(r)
    for i in range(_BB):
        o_ref[i] = jnp.concatenate(
            [rec[i * _H:(i + 1) * _H],
             rec[(_BB + i) * _H:(_BB + i + 1) * _H]], axis=1)


def kernel(x, mask_real, mask_imag, W1, b1, W2, b2):
    B = x.shape[0]
    xr = x.reshape(B, _H, 2 * _C)
    mr = jnp.tile(mask_real.reshape(1, _C), (1, _BB))
    mi = jnp.tile(mask_imag.reshape(1, _C), (1, _BB))
    b1r = b1.reshape(1, _C)
    b2r = b2.reshape(1, _C)
    const = lambda bs: pl.BlockSpec(bs, lambda b: (0,) * len(bs))
    out = pl.pallas_call(
        _body,
        grid=(B // _BB,),
        in_specs=[
            pl.BlockSpec((_BB, _H, 2 * _C), lambda b: (b, 0, 0)),
        ] + [const((2 * _KP, _H))] * 4 + [const((_H, _KP))] * 8 + [
            const((1, _NC)),
            const((1, _NC)),
            const((_C, _C)),
            const((1, _C)),
            const((_C, _C)),
            const((1, _C)),
        ],
        out_specs=pl.BlockSpec((_BB, _H, 2 * _C), lambda b: (b, 0, 0)),
        out_shape=jax.ShapeDtypeStruct((B, _H, 2 * _C), jnp.float32),
        compiler_params=pltpu.CompilerParams(
            dimension_semantics=("arbitrary",),
            vmem_limit_bytes=120 * 1024 * 1024,
        ),
    )(xr, jnp.asarray(_MEH), jnp.asarray(_MEL), jnp.asarray(_MOH),
      jnp.asarray(_MOL), jnp.asarray(_JCH), jnp.asarray(_JCL),
      jnp.asarray(_JSH), jnp.asarray(_JSL), jnp.asarray(_JCOH),
      jnp.asarray(_JCOL), jnp.asarray(_JSOH), jnp.asarray(_JSOL),
      mr, mi, W1, b1r, W2, b2r)
    return out.reshape(B, _N, _C)


# speedup vs baseline: 1.0295x; 1.0295x over previous
import numpy as np
import jax
import jax.numpy as jnp
from jax.experimental import pallas as pl
from jax.experimental.pallas import tpu as pltpu

_N = 2048
_H = 1024
_KP = 520
_C = 128
_BB = 2
_GRP = 2
_BT = _BB * _GRP
_NC = _BB * _C

_K_LO = 716

_IDX = np.float32(0.7) * np.float32(1024.0)
_LW = np.float32(np.float32(717.0) - _IDX)
_HW = np.float32(_IDX - np.float32(716.0))

_HI = jax.lax.Precision.HIGHEST


def _build_mats():
    k = np.arange(_KP, dtype=np.int64)[:, None]
    n = np.arange(_H, dtype=np.int64)[None, :]
    ang_e = (2.0 * np.pi / _N) * ((k * (2 * n)) % _N).astype(np.float64)
    ang_o = (2.0 * np.pi / _N) * ((k * (2 * n + 1)) % _N).astype(np.float64)
    valid = (k <= 512).astype(np.float64)
    me = np.concatenate([np.cos(ang_e) * valid, -np.sin(ang_e) * valid], 0)
    mo = np.concatenate([np.cos(ang_o) * valid, -np.sin(ang_o) * valid], 0)
    wgt = np.full((_KP, 1), 2.0 / _N)
    wgt[0] = 1.0 / _N
    wgt[512] = 1.0 / _N
    wgt[513:] = 0.0
    jc = (wgt * np.cos(ang_e)).T
    js = (-(wgt * np.sin(ang_e))).T
    jco = (wgt * np.cos(ang_o)).T
    jso = (-(wgt * np.sin(ang_o))).T
    f32 = np.float32
    return (me.astype(f32), mo.astype(f32), jc.astype(f32), js.astype(f32),
            jco.astype(f32), jso.astype(f32))


_ME, _MO, _JC, _JS, _JCO, _JSO = _build_mats()

import ml_dtypes as _mld


def _split_np(m):
    hi = m.astype(_mld.bfloat16)
    lo = (m - hi.astype(np.float32)).astype(_mld.bfloat16)
    return hi, lo


_MEH, _MEL = _split_np(_ME)
_MOH, _MOL = _split_np(_MO)
_JCH, _JCL = _split_np(_JC)
_JSH, _JSL = _split_np(_JS)
_JCOH, _JCOL = _split_np(_JCO)
_JSOH, _JSOL = _split_np(_JSO)


def _split(x):
    hi = x.astype(jnp.bfloat16)
    lo = (x - hi.astype(jnp.float32)).astype(jnp.bfloat16)
    return hi, lo


def _dot3(mh, ml, x):
    xh, xl = _split(x)
    f = jnp.float32
    return (jnp.dot(mh, xh, preferred_element_type=f) +
            (jnp.dot(mh, xl, preferred_element_type=f) +
             jnp.dot(ml, xh, preferred_element_type=f)))


def _dot3r(x, wh, wl):
    xh, xl = _split(x)
    f = jnp.float32
    return (jnp.dot(xh, wh, preferred_element_type=f) +
            (jnp.dot(xl, wh, preferred_element_type=f) +
             jnp.dot(xh, wl, preferred_element_type=f)))


def _body(x_ref, meh_ref, mel_ref, moh_ref, mol_ref,
          jch_ref, jcl_ref, jsh_ref, jsl_ref,
          jcoh_ref, jcol_ref, jsoh_ref, jsol_ref,
          mr_ref, mi_ref, w1_ref, b1_ref, w2_ref, b2_ref, o_ref):
    for g in range(_GRP):
        _group(g, x_ref, meh_ref, mel_ref, moh_ref, mol_ref,
               jch_ref, jcl_ref, jsh_ref, jsl_ref,
               jcoh_ref, jcol_ref, jsoh_ref, jsol_ref,
               mr_ref, mi_ref, w1_ref, b1_ref, w2_ref, b2_ref, o_ref)


def _group(g, x_ref, meh_ref, mel_ref, moh_ref, mol_ref,
           jch_ref, jcl_ref, jsh_ref, jsl_ref,
           jcoh_ref, jcol_ref, jsoh_ref, jsol_ref,
           mr_ref, mi_ref, w1_ref, b1_ref, w2_ref, b2_ref, o_ref):
    b0 = g * _BB
    xe = jnp.concatenate([x_ref[b0 + i, :, :_C] for i in range(_BB)], axis=1)
    xo = jnp.concatenate([x_ref[b0 + i, :, _C:] for i in range(_BB)], axis=1)
    e = _dot3(meh_ref[...], mel_ref[...], xe)
    op = _dot3(moh_ref[...], mol_ref[...], xo)
    xlo = e + op
    d = e - op
    xlr, xli = xlo[:_KP], xlo[_KP:]
    dr, di = d[:_KP], d[_KP:]

    riota = jax.lax.broadcasted_iota(jnp.int32, (_KP, _NC), 0)
    inf = jnp.float32(jnp.inf)
    mag_lo = jnp.sqrt(xlr * xlr + xli * xli)
    mag_d = jnp.sqrt(dr * dr + di * di)
    mag_lo = jnp.where(riota > 512, inf, mag_lo)
    mag_d = jnp.where(riota > 511, inf, mag_d)
    smag = jnp.concatenate([mag_lo, mag_d], axis=0)

    bits = jax.lax.bitcast_convert_type(smag, jnp.int32)
    ans = jnp.zeros((1, _NC), jnp.int32)
    for b in range(30, -1, -1):
        t = ans | (1 << b)
        cnt = jnp.sum((bits < t).astype(jnp.int32), axis=0, keepdims=True)
        ans = jnp.where(cnt <= _K_LO, t, ans)
    s_lo = jax.lax.bitcast_convert_type(ans, jnp.float32)
    cnt_le = jnp.sum((bits <= ans).astype(jnp.int32), axis=0, keepdims=True)
    above = jnp.where(smag > s_lo, smag, inf)
    s_hi = jnp.where(cnt_le >= _K_LO + 2, s_lo,
                     jnp.min(above, axis=0, keepdims=True))
    q = s_lo * _LW + s_hi * _HW

    tr = mr_ref[...]
    ti = mi_ref[...]
    cond_lo = mag_lo < q
    cond_d = mag_d < q
    clr = jnp.where(cond_lo, tr, xlr)
    cli = jnp.where(cond_lo, ti, xli)
    cdr = jnp.where(cond_d, tr, dr)
    cdi = jnp.where(cond_d, -ti, di)
    ar = clr + cdr
    ai = cli + cdi
    gr = clr - cdr
    gi = cli - cdi
    is512 = riota == 512
    zero = jnp.float32(0.0)
    ar = jnp.where(is512, 2.0 * clr, ar)
    ai = jnp.where(is512, zero, ai)
    gr = jnp.where(is512, zero, gr)
    gi = jnp.where(is512, 2.0 * cli, gi)

    dxe = (_dot3(jch_ref[...], jcl_ref[...], ar) +
           _dot3(jsh_ref[...], jsl_ref[...], ai))
    dxo = (_dot3(jcoh_ref[...], jcol_ref[...], gr) +
           _dot3(jsoh_ref[...], jsol_ref[...], gi))

    w1h, w1l = _split(w1_ref[...])
    w2h, w2l = _split(w2_ref[...])

    for i in range(_BB):
        sl = slice(i * _C, (i + 1) * _C)
        dcat = jnp.concatenate([dxe[:, sl], dxo[:, sl]], axis=0)
        h = _dot3r(dcat, w1h, w1l) + b1_ref[...]
        h = 0.5 * h * (1.0 + jax.lax.erf(h * np.float32(1.0 / np.sqrt(2.0))))
        r = _dot3r(h, w2h, w2l) + b2_ref[...]
        rec = jax.nn.sigmoid(r)
        o_ref[b0 + i] = jnp.concatenate([rec[:_H], rec[_H:]], axis=1)


def kernel(x, mask_real, mask_imag, W1, b1, W2, b2):
    B = x.shape[0]
    xr = x.reshape(B, _H, 2 * _C)
    mr = jnp.tile(mask_real.reshape(1, _C), (1, _BB))
    mi = jnp.tile(mask_imag.reshape(1, _C), (1, _BB))
    b1r = b1.reshape(1, _C)
    b2r = b2.reshape(1, _C)
    const = lambda bs: pl.BlockSpec(bs, lambda b: (0,) * len(bs))
    out = pl.pallas_call(
        _body,
        grid=(B // _BT,),
        in_specs=[
            pl.BlockSpec((_BT, _H, 2 * _C), lambda b: (b, 0, 0)),
        ] + [const((2 * _KP, _H))] * 4 + [const((_H, _KP))] * 8 + [
            const((1, _NC)),
            const((1, _NC)),
            const((_C, _C)),
            const((1, _C)),
            const((_C, _C)),
            const((1, _C)),
        ],
        out_specs=pl.BlockSpec((_BT, _H, 2 * _C), lambda b: (b, 0, 0)),
        out_shape=jax.ShapeDtypeStruct((B, _H, 2 * _C), jnp.float32),
        compiler_params=pltpu.CompilerParams(
            dimension_semantics=("arbitrary",),
            vmem_limit_bytes=120 * 1024 * 1024,
        ),
    )(xr, jnp.asarray(_MEH), jnp.asarray(_MEL), jnp.asarray(_MOH),
      jnp.asarray(_MOL), jnp.asarray(_JCH), jnp.asarray(_JCL),
      jnp.asarray(_JSH), jnp.asarray(_JSL), jnp.asarray(_JCOH),
      jnp.asarray(_JCOL), jnp.asarray(_JSOH), jnp.asarray(_JSOL),
      mr, mi, W1, b1r, W2, b2r)
    return out.reshape(B, _N, _C)


# speedup vs baseline: 1.1472x; 1.1143x over previous
import numpy as np
import jax
import jax.numpy as jnp
from jax.experimental import pallas as pl
from jax.experimental.pallas import tpu as pltpu

_N = 2048
_H = 1024
_KP = 520
_C = 128
_BB = 2
_GRP = 2
_BT = _BB * _GRP
_NC = _BB * _C

_K_LO = 716

_IDX = np.float32(0.7) * np.float32(1024.0)
_LW = np.float32(np.float32(717.0) - _IDX)
_HW = np.float32(_IDX - np.float32(716.0))

_HI = jax.lax.Precision.HIGHEST


def _build_mats():
    k = np.arange(_KP, dtype=np.int64)[:, None]
    n = np.arange(_H, dtype=np.int64)[None, :]
    ang_e = (2.0 * np.pi / _N) * ((k * (2 * n)) % _N).astype(np.float64)
    ang_o = (2.0 * np.pi / _N) * ((k * (2 * n + 1)) % _N).astype(np.float64)
    valid = (k <= 512).astype(np.float64)
    me = np.concatenate([np.cos(ang_e) * valid, -np.sin(ang_e) * valid], 0)
    mo = np.concatenate([np.cos(ang_o) * valid, -np.sin(ang_o) * valid], 0)
    wgt = np.full((_KP, 1), 2.0 / _N)
    wgt[0] = 1.0 / _N
    wgt[512] = 1.0 / _N
    wgt[513:] = 0.0
    jc = (wgt * np.cos(ang_e)).T
    js = (-(wgt * np.sin(ang_e))).T
    jco = (wgt * np.cos(ang_o)).T
    jso = (-(wgt * np.sin(ang_o))).T
    f32 = np.float32
    return (me.astype(f32), mo.astype(f32), jc.astype(f32), js.astype(f32),
            jco.astype(f32), jso.astype(f32))


_ME, _MO, _JC, _JS, _JCO, _JSO = _build_mats()

import ml_dtypes as _mld


def _split_np(m):
    hi = m.astype(_mld.bfloat16)
    lo = (m - hi.astype(np.float32)).astype(_mld.bfloat16)
    return hi, lo


_MEH, _MEL = _split_np(_ME)
_MOH, _MOL = _split_np(_MO)
_JCH, _JCL = _split_np(_JC)
_JSH, _JSL = _split_np(_JS)
_JCOH, _JCOL = _split_np(_JCO)
_JSOH, _JSOL = _split_np(_JSO)


def _split(x):
    hi = x.astype(jnp.bfloat16)
    lo = (x - hi.astype(jnp.float32)).astype(jnp.bfloat16)
    return hi, lo


def _dot3(mh, ml, x):
    xh, xl = _split(x)
    f = jnp.float32
    return (jnp.dot(mh, xh, preferred_element_type=f) +
            (jnp.dot(mh, xl, preferred_element_type=f) +
             jnp.dot(ml, xh, preferred_element_type=f)))


def _dot3r(x, wh, wl):
    xh, xl = _split(x)
    f = jnp.float32
    return (jnp.dot(xh, wh, preferred_element_type=f) +
            (jnp.dot(xl, wh, preferred_element_type=f) +
             jnp.dot(xh, wl, preferred_element_type=f)))


def _rsum16(a):
    rows = a.shape[0]
    parts = [a[i * 16:(i + 1) * 16] for i in range(rows // 16)]
    while len(parts) > 1:
        nxt = [parts[i] + parts[i + 1] for i in range(0, len(parts) - 1, 2)]
        if len(parts) % 2:
            nxt.append(parts[-1])
        parts = nxt
    return jnp.sum(parts[0].astype(jnp.int32), axis=0, keepdims=True)


def _body(x_ref, meh_ref, mel_ref, moh_ref, mol_ref,
          jch_ref, jcl_ref, jsh_ref, jsl_ref,
          jcoh_ref, jcol_ref, jsoh_ref, jsol_ref,
          mr_ref, mi_ref, w1_ref, b1_ref, w2_ref, b2_ref, o_ref):
    for g in range(_GRP):
        _group(g, x_ref, meh_ref, mel_ref, moh_ref, mol_ref,
               jch_ref, jcl_ref, jsh_ref, jsl_ref,
               jcoh_ref, jcol_ref, jsoh_ref, jsol_ref,
               mr_ref, mi_ref, w1_ref, b1_ref, w2_ref, b2_ref, o_ref)


def _group(g, x_ref, meh_ref, mel_ref, moh_ref, mol_ref,
           jch_ref, jcl_ref, jsh_ref, jsl_ref,
           jcoh_ref, jcol_ref, jsoh_ref, jsol_ref,
           mr_ref, mi_ref, w1_ref, b1_ref, w2_ref, b2_ref, o_ref):
    b0 = g * _BB
    xe = jnp.concatenate([x_ref[b0 + i, :, :_C] for i in range(_BB)], axis=1)
    xo = jnp.concatenate([x_ref[b0 + i, :, _C:] for i in range(_BB)], axis=1)
    e = _dot3(meh_ref[...], mel_ref[...], xe)
    op = _dot3(moh_ref[...], mol_ref[...], xo)
    xlo = e + op
    d = e - op
    xlr, xli = xlo[:_KP], xlo[_KP:]
    dr, di = d[:_KP], d[_KP:]

    riota = jax.lax.broadcasted_iota(jnp.int32, (_KP, _NC), 0)
    inf = jnp.float32(jnp.inf)
    mag_lo = jnp.sqrt(xlr * xlr + xli * xli)
    mag_d = jnp.sqrt(dr * dr + di * di)
    mag_lo = jnp.where(riota > 512, inf, mag_lo)
    mag_d = jnp.where(riota > 511, inf, mag_d)
    smag = jnp.concatenate([mag_lo, mag_d], axis=0)

    bits = jax.lax.bitcast_convert_type(smag, jnp.int32)
    h16 = (bits >> 16).astype(jnp.int16)
    l16 = ((bits & 0xFFFF) - 32768).astype(jnp.int16)
    i16 = jnp.int16
    ansh = jnp.zeros((1, _NC), jnp.int32)
    for b in range(14, -1, -1):
        t = ansh | (1 << b)
        cnt = _rsum16((h16 < t.astype(i16)).astype(i16))
        ansh = jnp.where(cnt <= _K_LO, t, ansh)
    ansh16 = ansh.astype(i16)
    eqm = (h16 == ansh16).astype(i16)
    cless = _rsum16((h16 < ansh16).astype(i16))
    kk = _K_LO - cless
    ansl = jnp.zeros((1, _NC), jnp.int32)
    for b in range(15, -1, -1):
        t_u = ansl | (1 << b)
        t_s = (t_u - 32768).astype(i16)
        cnt = _rsum16(eqm & (l16 < t_s).astype(i16))
        ansl = jnp.where(cnt <= kk, t_u, ansl)
    ans = (ansh << 16) | ansl
    s_lo = jax.lax.bitcast_convert_type(ans, jnp.float32)
    cnt_le = jnp.sum((bits <= ans).astype(jnp.int32), axis=0, keepdims=True)
    above = jnp.where(smag > s_lo, smag, inf)
    s_hi = jnp.where(cnt_le >= _K_LO + 2, s_lo,
                     jnp.min(above, axis=0, keepdims=True))
    q = s_lo * _LW + s_hi * _HW

    tr = mr_ref[...]
    ti = mi_ref[...]
    cond_lo = mag_lo < q
    cond_d = mag_d < q
    clr = jnp.where(cond_lo, tr, xlr)
    cli = jnp.where(cond_lo, ti, xli)
    cdr = jnp.where(cond_d, tr, dr)
    cdi = jnp.where(cond_d, -ti, di)
    ar = clr + cdr
    ai = cli + cdi
    gr = clr - cdr
    gi = cli - cdi
    is512 = riota == 512
    zero = jnp.float32(0.0)
    ar = jnp.where(is512, 2.0 * clr, ar)
    ai = jnp.where(is512, zero, ai)
    gr = jnp.where(is512, zero, gr)
    gi = jnp.where(is512, 2.0 * cli, gi)

    dxe = (_dot3(jch_ref[...], jcl_ref[...], ar) +
           _dot3(jsh_ref[...], jsl_ref[...], ai))
    dxo = (_dot3(jcoh_ref[...], jcol_ref[...], gr) +
           _dot3(jsoh_ref[...], jsol_ref[...], gi))

    w1h, w1l = _split(w1_ref[...])
    w2h, w2l = _split(w2_ref[...])

    for i in range(_BB):
        sl = slice(i * _C, (i + 1) * _C)
        dcat = jnp.concatenate([dxe[:, sl], dxo[:, sl]], axis=0)
        h = _dot3r(dcat, w1h, w1l) + b1_ref[...]
        h = 0.5 * h * (1.0 + jax.lax.erf(h * np.float32(1.0 / np.sqrt(2.0))))
        r = _dot3r(h, w2h, w2l) + b2_ref[...]
        rec = jax.nn.sigmoid(r)
        o_ref[b0 + i] = jnp.concatenate([rec[:_H], rec[_H:]], axis=1)


def kernel(x, mask_real, mask_imag, W1, b1, W2, b2):
    B = x.shape[0]
    xr = x.reshape(B, _H, 2 * _C)
    mr = jnp.tile(mask_real.reshape(1, _C), (1, _BB))
    mi = jnp.tile(mask_imag.reshape(1, _C), (1, _BB))
    b1r = b1.reshape(1, _C)
    b2r = b2.reshape(1, _C)
    const = lambda bs: pl.BlockSpec(bs, lambda b: (0,) * len(bs))
    out = pl.pallas_call(
        _body,
        grid=(B // _BT,),
        in_specs=[
            pl.BlockSpec((_BT, _H, 2 * _C), lambda b: (b, 0, 0)),
        ] + [const((2 * _KP, _H))] * 4 + [const((_H, _KP))] * 8 + [
            const((1, _NC)),
            const((1, _NC)),
            const((_C, _C)),
            const((1, _C)),
            const((_C, _C)),
            const((1, _C)),
        ],
        out_specs=pl.BlockSpec((_BT, _H, 2 * _C), lambda b: (b, 0, 0)),
        out_shape=jax.ShapeDtypeStruct((B, _H, 2 * _C), jnp.float32),
        compiler_params=pltpu.CompilerParams(
            dimension_semantics=("arbitrary",),
            vmem_limit_bytes=120 * 1024 * 1024,
        ),
    )(xr, jnp.asarray(_MEH), jnp.asarray(_MEL), jnp.asarray(_MOH),
      jnp.asarray(_MOL), jnp.asarray(_JCH), jnp.asarray(_JCL),
      jnp.asarray(_JSH), jnp.asarray(_JSL), jnp.asarray(_JCOH),
      jnp.asarray(_JCOL), jnp.asarray(_JSOH), jnp.asarray(_JSOL),
      mr, mi, W1, b1r, W2, b2r)
    return out.reshape(B, _N, _C)


# speedup vs baseline: 1.1472x; 1.0000x over previous
import numpy as np
import jax
import jax.numpy as jnp
from jax.experimental import pallas as pl
from jax.experimental.pallas import tpu as pltpu

_N = 2048
_H = 1024
_KP = 520
_C = 128
_BB = 2
_GRP = 2
_BT = _BB * _GRP
_NC = _BB * _C

_K_LO = 716

_IDX = np.float32(0.7) * np.float32(1024.0)
_LW = np.float32(np.float32(717.0) - _IDX)
_HW = np.float32(_IDX - np.float32(716.0))

_HI = jax.lax.Precision.HIGHEST


def _build_mats():
    k = np.arange(_KP, dtype=np.int64)[:, None]
    n = np.arange(_H, dtype=np.int64)[None, :]
    ang_e = (2.0 * np.pi / _N) * ((k * (2 * n)) % _N).astype(np.float64)
    ang_o = (2.0 * np.pi / _N) * ((k * (2 * n + 1)) % _N).astype(np.float64)
    valid = (k <= 512).astype(np.float64)
    me = np.concatenate([np.cos(ang_e) * valid, -np.sin(ang_e) * valid], 0)
    mo = np.concatenate([np.cos(ang_o) * valid, -np.sin(ang_o) * valid], 0)
    wgt = np.full((_KP, 1), 2.0 / _N)
    wgt[0] = 1.0 / _N
    wgt[512] = 1.0 / _N
    wgt[513:] = 0.0
    jc = (wgt * np.cos(ang_e)).T
    js = (-(wgt * np.sin(ang_e))).T
    jco = (wgt * np.cos(ang_o)).T
    jso = (-(wgt * np.sin(ang_o))).T
    f32 = np.float32
    return (me.astype(f32), mo.astype(f32), jc.astype(f32), js.astype(f32),
            jco.astype(f32), jso.astype(f32))


_ME, _MO, _JC, _JS, _JCO, _JSO = _build_mats()

import ml_dtypes as _mld


def _split_np(m):
    hi = m.astype(_mld.bfloat16)
    lo = (m - hi.astype(np.float32)).astype(_mld.bfloat16)
    return hi, lo


_MEH, _MEL = _split_np(_ME)
_MOH, _MOL = _split_np(_MO)
_JCH, _JCL = _split_np(_JC)
_JSH, _JSL = _split_np(_JS)
_JCOH, _JCOL = _split_np(_JCO)
_JSOH, _JSOL = _split_np(_JSO)


def _split(x):
    hi = x.astype(jnp.bfloat16)
    lo = (x - hi.astype(jnp.float32)).astype(jnp.bfloat16)
    return hi, lo


def _dot3(mh, ml, x):
    xh, xl = _split(x)
    f = jnp.float32
    return (jnp.dot(mh, xh, preferred_element_type=f) +
            (jnp.dot(mh, xl, preferred_element_type=f) +
             jnp.dot(ml, xh, preferred_element_type=f)))


def _dot3r(x, wh, wl):
    xh, xl = _split(x)
    f = jnp.float32
    return (jnp.dot(xh, wh, preferred_element_type=f) +
            (jnp.dot(xl, wh, preferred_element_type=f) +
             jnp.dot(xh, wl, preferred_element_type=f)))


def _rsum16(a):
    rows = a.shape[0]
    parts = [a[i * 16:(i + 1) * 16] for i in range(rows // 16)]
    while len(parts) > 1:
        nxt = [parts[i] + parts[i + 1] for i in range(0, len(parts) - 1, 2)]
        if len(parts) % 2:
            nxt.append(parts[-1])
        parts = nxt
    return jnp.sum(parts[0].astype(jnp.int32), axis=0, keepdims=True)


def _body(x_ref, meh_ref, mel_ref, moh_ref, mol_ref,
          jch_ref, jcl_ref, jsh_ref, jsl_ref,
          jcoh_ref, jcol_ref, jsoh_ref, jsol_ref,
          mr_ref, mi_ref, w1_ref, b1_ref, w2_ref, b2_ref, o_ref):
    for g in range(_GRP):
        _group(g, x_ref, meh_ref, mel_ref, moh_ref, mol_ref,
               jch_ref, jcl_ref, jsh_ref, jsl_ref,
               jcoh_ref, jcol_ref, jsoh_ref, jsol_ref,
               mr_ref, mi_ref, w1_ref, b1_ref, w2_ref, b2_ref, o_ref)


def _group(g, x_ref, meh_ref, mel_ref, moh_ref, mol_ref,
           jch_ref, jcl_ref, jsh_ref, jsl_ref,
           jcoh_ref, jcol_ref, jsoh_ref, jsol_ref,
           mr_ref, mi_ref, w1_ref, b1_ref, w2_ref, b2_ref, o_ref):
    b0 = g * _BB
    xe = jnp.concatenate([x_ref[b0 + i, :, :_C] for i in range(_BB)], axis=1)
    xo = jnp.concatenate([x_ref[b0 + i, :, _C:] for i in range(_BB)], axis=1)
    e = _dot3(meh_ref[...], mel_ref[...], xe)
    op = _dot3(moh_ref[...], mol_ref[...], xo)
    xlo = e + op
    d = e - op
    xlr, xli = xlo[:_KP], xlo[_KP:]
    dr, di = d[:_KP], d[_KP:]

    riota = jax.lax.broadcasted_iota(jnp.int32, (_KP, _NC), 0)
    inf = jnp.float32(jnp.inf)
    mag_lo = jnp.sqrt(xlr * xlr + xli * xli)
    mag_d = jnp.sqrt(dr * dr + di * di)
    mag_lo = jnp.where(riota > 512, inf, mag_lo)
    mag_d = jnp.where(riota > 511, inf, mag_d)
    smag = jnp.concatenate([mag_lo, mag_d], axis=0)

    bits = jax.lax.bitcast_convert_type(smag, jnp.int32)
    h16 = (bits >> 16).astype(jnp.int16)
    l16 = ((bits & 0xFFFF) - 32768).astype(jnp.int16)
    i16 = jnp.int16
    ansh = jnp.zeros((1, _NC), jnp.int32)
    for b in range(14, -1, -1):
        t = ansh | (1 << b)
        cnt = _rsum16((h16 < t.astype(i16)).astype(i16))
        ansh = jnp.where(cnt <= _K_LO, t, ansh)
    ansh16 = ansh.astype(i16)
    eqm = (h16 == ansh16).astype(i16)
    cless = _rsum16((h16 < ansh16).astype(i16))
    kk = _K_LO - cless
    ansl = jnp.zeros((1, _NC), jnp.int32)
    for b in range(15, -1, -1):
        t_u = ansl | (1 << b)
        t_s = (t_u - 32768).astype(i16)
        cnt = _rsum16(eqm & (l16 < t_s).astype(i16))
        ansl = jnp.where(cnt <= kk, t_u, ansl)
    ans = (ansh << 16) | ansl
    s_lo = jax.lax.bitcast_convert_type(ans, jnp.float32)
    cnt_le = jnp.sum((bits <= ans).astype(jnp.int32), axis=0, keepdims=True)
    above = jnp.where(smag > s_lo, smag, inf)
    s_hi = jnp.where(cnt_le >= _K_LO + 2, s_lo,
                     jnp.min(above, axis=0, keepdims=True))
    q = s_lo * _LW + s_hi * _HW

    tr = mr_ref[...]
    ti = mi_ref[...]
    cond_lo = mag_lo < q
    cond_d = mag_d < q
    clr = jnp.where(cond_lo, tr, xlr)
    cli = jnp.where(cond_lo, ti, xli)
    cdr = jnp.where(cond_d, tr, dr)
    cdi = jnp.where(cond_d, -ti, di)
    ar = clr + cdr
    ai = cli + cdi
    gr = clr - cdr
    gi = cli - cdi
    is512 = riota == 512
    zero = jnp.float32(0.0)
    ar = jnp.where(is512, 2.0 * clr, ar)
    ai = jnp.where(is512, zero, ai)
    gr = jnp.where(is512, zero, gr)
    gi = jnp.where(is512, 2.0 * cli, gi)

    dxe = (_dot3(jch_ref[...], jcl_ref[...], ar) +
           _dot3(jsh_ref[...], jsl_ref[...], ai))
    dxo = (_dot3(jcoh_ref[...], jcol_ref[...], gr) +
           _dot3(jsoh_ref[...], jsol_ref[...], gi))

    w1h, w1l = _split(w1_ref[...])
    w2h, w2l = _split(w2_ref[...])

    for i in range(_BB):
        sl = slice(i * _C, (i + 1) * _C)
        dcat = jnp.concatenate([dxe[:, sl], dxo[:, sl]], axis=0)
        h = _dot3r(dcat, w1h, w1l) + b1_ref[...]
        h = 0.5 * h * (1.0 + jax.lax.erf(h * np.float32(1.0 / np.sqrt(2.0))))
        r = _dot3r(h, w2h, w2l) + b2_ref[...]
        rec = jax.nn.sigmoid(r)
        o_ref[b0 + i] = jnp.concatenate([rec[:_H], rec[_H:]], axis=1)


def kernel(x, mask_real, mask_imag, W1, b1, W2, b2):
    B = x.shape[0]
    xr = x.reshape(B, _H, 2 * _C)
    mr = jnp.tile(mask_real.reshape(1, _C), (1, _BB))
    mi = jnp.tile(mask_imag.reshape(1, _C), (1, _BB))
    b1r = b1.reshape(1, _C)
    b2r = b2.reshape(1, _C)
    const = lambda bs: pl.BlockSpec(bs, lambda b: (0,) * len(bs))
    out = pl.pallas_call(
        _body,
        grid=(B // _BT,),
        in_specs=[
            pl.BlockSpec((_BT, _H, 2 * _C), lambda b: (b, 0, 0)),
        ] + [const((2 * _KP, _H))] * 4 + [const((_H, _KP))] * 8 + [
            const((1, _NC)),
            const((1, _NC)),
            const((_C, _C)),
            const((1, _C)),
            const((_C, _C)),
            const((1, _C)),
        ],
        out_specs=pl.BlockSpec((_BT, _H, 2 * _C), lambda b: (b, 0, 0)),
        out_shape=jax.ShapeDtypeStruct((B, _H, 2 * _C), jnp.float32),
        compiler_params=pltpu.CompilerParams(
            dimension_semantics=("parallel",),
            vmem_limit_bytes=120 * 1024 * 1024,
        ),
    )(xr, jnp.asarray(_MEH), jnp.asarray(_MEL), jnp.asarray(_MOH),
      jnp.asarray(_MOL), jnp.asarray(_JCH), jnp.asarray(_JCL),
      jnp.asarray(_JSH), jnp.asarray(_JSL), jnp.asarray(_JCOH),
      jnp.asarray(_JCOL), jnp.asarray(_JSOH), jnp.asarray(_JSOL),
      mr, mi, W1, b1r, W2, b2r)
    return out.reshape(B, _N, _C)
